# baseline (speedup 1.0000x reference)
import jax
from jax.experimental import pallas as pl


def _copy_kernel(x_ref, o_ref):
    o_ref[...] = x_ref[...]


def kernel(x, adj, embed_table):
    del adj, embed_table
    rows, cols = x.shape
    n = rows * cols
    xv = x.reshape(n // 128, 128)
    out = pl.pallas_call(
        _copy_kernel,
        out_shape=jax.ShapeDtypeStruct(xv.shape, xv.dtype),
    )(xv)
    return out.reshape(rows, cols)
